# baseline (speedup 1.0000x reference)
import jax
import jax.numpy as jnp
from jax import lax
from jax.experimental import pallas as pl
from jax.experimental.pallas import tpu as pltpu

B = 128
V = 100000
RB = 8
NRB = B // RB
INT_MAX = 2**31 - 1


def _body(pv_ref, log_ref, pen_ref, idx_ref, out_ref):
    pen = pen_ref[...]
    scaled = log_ref[...] * pen
    col = lax.broadcasted_iota(jnp.int32, (RB, V), 1)
    bmax = jnp.max(scaled, axis=1, keepdims=True)
    cand = jnp.where(scaled == bmax, col, jnp.int32(INT_MAX))
    barg = jnp.min(cand, axis=1, keepdims=True)
    idx_ref[...] = barg
    hit = col == barg
    out_ref[...] = jnp.where(hit, pen * pv_ref[0, 0], pen)


def kernel(logits, repeat_penality, penality_value):
    idx, pen_out = pl.pallas_call(
        _body,
        grid=(NRB,),
        in_specs=[
            pl.BlockSpec(memory_space=pltpu.SMEM),
            pl.BlockSpec((RB, V), lambda i: (i, 0)),
            pl.BlockSpec((RB, V), lambda i: (i, 0)),
        ],
        out_specs=[
            pl.BlockSpec((RB, 1), lambda i: (i, 0)),
            pl.BlockSpec((RB, V), lambda i: (i, 0)),
        ],
        out_shape=[
            jax.ShapeDtypeStruct((B, 1), jnp.int32),
            jax.ShapeDtypeStruct((B, V), jnp.float32),
        ],
        compiler_params=pltpu.CompilerParams(
            dimension_semantics=("parallel",),
        ),
    )(penality_value.reshape(1, 1), logits, repeat_penality)
    return idx, pen_out


# speedup vs baseline: 1.0214x; 1.0214x over previous
import jax
import jax.numpy as jnp
from jax import lax
from jax.experimental import pallas as pl
from jax.experimental.pallas import tpu as pltpu

B = 128
V = 100000
RB = 8
NC = B // RB
DEPTH = 2
INT_MAX = 2**31 - 1


def _body(pv_ref, log_hbm, pen_hbm, idx_hbm, out_hbm,
          log_v, pen_v, out_v, idx_v, in_sems, out_sems, idx_sem):
    def in_copies(k, s):
        return (
            pltpu.make_async_copy(
                log_hbm.at[pl.ds(k * RB, RB), :], log_v.at[s], in_sems.at[0, s]
            ),
            pltpu.make_async_copy(
                pen_hbm.at[pl.ds(k * RB, RB), :], pen_v.at[s], in_sems.at[1, s]
            ),
        )

    def out_copy(k, s):
        return pltpu.make_async_copy(
            out_v.at[s], out_hbm.at[pl.ds(k * RB, RB), :], out_sems.at[s]
        )

    for s in range(DEPTH):
        for c in in_copies(s, s):
            c.start()

    pv = pv_ref[0, 0]
    col = lax.broadcasted_iota(jnp.int32, (RB, V), 1)

    def loop(o, _):
        for s in range(DEPTH):
            k = o * DEPTH + s
            for c in in_copies(k, s):
                c.wait()

            @pl.when(k >= DEPTH)
            def _():
                out_copy(k - DEPTH, s).wait()

            pen = pen_v[s]
            scaled = log_v[s] * pen
            bmax = jnp.max(scaled, axis=1, keepdims=True)
            cand = jnp.where(scaled == bmax, col, jnp.int32(INT_MAX))
            barg = jnp.min(cand, axis=1, keepdims=True)
            idx_v[pl.ds(k * RB, RB), :] = barg
            hit = col == barg
            out_v[s] = jnp.where(hit, pen * pv, pen)
            out_copy(k, s).start()

            @pl.when(k + DEPTH < NC)
            def _():
                for c in in_copies(k + DEPTH, s):
                    c.start()

        return _

    lax.fori_loop(0, NC // DEPTH, loop, None)

    for s in range(DEPTH):
        out_copy(NC - DEPTH + s, s).wait()

    pltpu.make_async_copy(idx_v, idx_hbm, idx_sem).start()
    pltpu.make_async_copy(idx_v, idx_hbm, idx_sem).wait()


def kernel(logits, repeat_penality, penality_value):
    idx, pen_out = pl.pallas_call(
        _body,
        in_specs=[
            pl.BlockSpec(memory_space=pltpu.SMEM),
            pl.BlockSpec(memory_space=pl.ANY),
            pl.BlockSpec(memory_space=pl.ANY),
        ],
        out_specs=[
            pl.BlockSpec(memory_space=pl.ANY),
            pl.BlockSpec(memory_space=pl.ANY),
        ],
        out_shape=[
            jax.ShapeDtypeStruct((B, 1), jnp.int32),
            jax.ShapeDtypeStruct((B, V), jnp.float32),
        ],
        scratch_shapes=[
            pltpu.VMEM((DEPTH, RB, V), jnp.float32),
            pltpu.VMEM((DEPTH, RB, V), jnp.float32),
            pltpu.VMEM((DEPTH, RB, V), jnp.float32),
            pltpu.VMEM((B, 1), jnp.int32),
            pltpu.SemaphoreType.DMA((2, DEPTH)),
            pltpu.SemaphoreType.DMA((DEPTH,)),
            pltpu.SemaphoreType.DMA,
        ],
    )(penality_value.reshape(1, 1), logits, repeat_penality)
    return idx, pen_out


# speedup vs baseline: 2.5598x; 2.5061x over previous
import functools

import jax
import jax.numpy as jnp
from jax import lax
from jax.experimental import pallas as pl
from jax.experimental.pallas import tpu as pltpu
from jax.experimental.pallas import tpu_sc as plsc

B = 128
V = 100000
VBS = 4000
NB = V // VBS
INT_MAX = 2**31 - 1
L = 16


def _stream_body(log_ref, pen_ref, idx_ref, out_ref, maxv, argv):
    j = pl.program_id(0)
    pen = pen_ref[...]
    out_ref[...] = pen
    scaled = log_ref[...] * pen
    rows = lax.broadcasted_iota(jnp.int32, (VBS, B), 0) + j * VBS
    bmax = jnp.max(scaled, axis=0, keepdims=True)
    cand = jnp.where(scaled == bmax, rows, jnp.int32(INT_MAX))
    bargm = jnp.min(cand, axis=0, keepdims=True)

    @pl.when(j == 0)
    def _():
        maxv[0:1, :] = bmax
        argv[0:1, :] = bargm

    @pl.when(j > 0)
    def _():
        upd = bmax > maxv[0:1, :]
        maxv[0:1, :] = jnp.where(upd, bmax, maxv[0:1, :])
        argv[0:1, :] = jnp.where(upd, bargm, argv[0:1, :])

    @pl.when(j == NB - 1)
    def _():
        idx_ref[...] = jnp.broadcast_to(argv[0:1, :], (8, B))


def _stream_pass(log_t, pen_t):
    return pl.pallas_call(
        _stream_body,
        grid=(NB,),
        in_specs=[
            pl.BlockSpec((VBS, B), lambda j: (j, 0)),
            pl.BlockSpec((VBS, B), lambda j: (j, 0)),
        ],
        out_specs=[
            pl.BlockSpec((8, B), lambda j: (0, 0)),
            pl.BlockSpec((VBS, B), lambda j: (j, 0)),
        ],
        out_shape=[
            jax.ShapeDtypeStruct((8, B), jnp.int32),
            jax.ShapeDtypeStruct((V, B), jnp.float32),
        ],
        scratch_shapes=[
            pltpu.VMEM((8, B), jnp.float32),
            pltpu.VMEM((8, B), jnp.int32),
        ],
        compiler_params=pltpu.CompilerParams(
            dimension_semantics=("arbitrary",),
        ),
    )(log_t, pen_t)


def _sc_fixup_body(pen_ref, idx_hbm, pv_hbm, idx_v, flat_v, vals_v, pv_v, sem):
    cid = lax.axis_index("c")
    sid = lax.axis_index("s")

    @pl.when(jnp.logical_and(cid == 0, sid == 0))
    def _():
        pltpu.sync_copy(idx_hbm, idx_v)
        pltpu.sync_copy(pv_hbm, pv_v)
        for k in range(B // L):
            lanes = lax.iota(jnp.int32, L) + (k * L)
            flat_v[0, pl.ds(k * L, L)] = idx_v[pl.ds(k * L, L)] * B + lanes
        pltpu.async_copy(pen_ref.at[flat_v.at[0]], vals_v, sem).wait()
        for k in range(B // L):
            vals_v[pl.ds(k * L, L)] = vals_v[pl.ds(k * L, L)] * pv_v[...]
        pltpu.async_copy(vals_v, pen_ref.at[flat_v.at[0]], sem).wait()


@functools.cache
def _make_sc_fixup():
    mesh = plsc.VectorSubcoreMesh(core_axis_name="c", subcore_axis_name="s")
    return pl.kernel(
        _sc_fixup_body,
        out_type=(),
        mesh=mesh,
        scratch_types=[
            pltpu.VMEM((B,), jnp.int32),
            pltpu.VMEM((1, B), jnp.int32),
            pltpu.VMEM((B,), jnp.float32),
            pltpu.VMEM((L,), jnp.float32),
            pltpu.SemaphoreType.DMA,
        ],
    )


def kernel(logits, repeat_penality, penality_value):
    log_t = logits.T
    pen_t = repeat_penality.T
    idx8, out_t = _stream_pass(log_t, pen_t)
    idx = idx8[0]
    pen_flat_ref = jax.new_ref(out_t.reshape(V * B))
    pv16 = jnp.full((L,), penality_value, dtype=jnp.float32)
    _make_sc_fixup()(pen_flat_ref, idx, pv16)
    return idx.reshape(B, 1), pen_flat_ref[...].reshape(V, B).T


# speedup vs baseline: 2.6743x; 1.0447x over previous
import functools

import jax
import jax.numpy as jnp
from jax import lax
from jax.experimental import pallas as pl
from jax.experimental.pallas import tpu as pltpu
from jax.experimental.pallas import tpu_sc as plsc

B = 128
V = 100000
VBS = 10000
NB = V // VBS
INT_MAX = 2**31 - 1
L = 16


def _stream_body(log_ref, pen_ref, idx_ref, out_ref, maxv, argv):
    j = pl.program_id(0)
    pen = pen_ref[...]
    out_ref[...] = pen
    scaled = log_ref[...] * pen
    rows = lax.broadcasted_iota(jnp.int32, (VBS, B), 0) + j * VBS
    bmax = jnp.max(scaled, axis=0, keepdims=True)
    cand = jnp.where(scaled == bmax, rows, jnp.int32(INT_MAX))
    bargm = jnp.min(cand, axis=0, keepdims=True)

    @pl.when(j == 0)
    def _():
        maxv[0:1, :] = bmax
        argv[0:1, :] = bargm

    @pl.when(j > 0)
    def _():
        upd = bmax > maxv[0:1, :]
        maxv[0:1, :] = jnp.where(upd, bmax, maxv[0:1, :])
        argv[0:1, :] = jnp.where(upd, bargm, argv[0:1, :])

    @pl.when(j == NB - 1)
    def _():
        idx_ref[...] = jnp.broadcast_to(argv[0:1, :], (8, B))


def _stream_pass(log_t, pen_t):
    return pl.pallas_call(
        _stream_body,
        grid=(NB,),
        in_specs=[
            pl.BlockSpec((VBS, B), lambda j: (j, 0)),
            pl.BlockSpec((VBS, B), lambda j: (j, 0)),
        ],
        out_specs=[
            pl.BlockSpec((8, B), lambda j: (0, 0)),
            pl.BlockSpec((VBS, B), lambda j: (j, 0)),
        ],
        out_shape=[
            jax.ShapeDtypeStruct((8, B), jnp.int32),
            jax.ShapeDtypeStruct((V, B), jnp.float32),
        ],
        scratch_shapes=[
            pltpu.VMEM((8, B), jnp.float32),
            pltpu.VMEM((8, B), jnp.int32),
        ],
        compiler_params=pltpu.CompilerParams(
            dimension_semantics=("arbitrary",),
        ),
    )(log_t, pen_t)


def _sc_fixup_body(pen_ref, idx_hbm, pv_hbm, idx_v, flat_v, vals_v, pv_v, sem):
    cid = lax.axis_index("c")
    sid = lax.axis_index("s")

    @pl.when(jnp.logical_and(cid == 0, sid == 0))
    def _():
        pltpu.sync_copy(idx_hbm, idx_v)
        pltpu.sync_copy(pv_hbm, pv_v)
        for k in range(B // L):
            lanes = lax.iota(jnp.int32, L) + (k * L)
            flat_v[0, pl.ds(k * L, L)] = idx_v[pl.ds(k * L, L)] * B + lanes
        pltpu.async_copy(pen_ref.at[flat_v.at[0]], vals_v, sem).wait()
        for k in range(B // L):
            vals_v[pl.ds(k * L, L)] = vals_v[pl.ds(k * L, L)] * pv_v[...]
        pltpu.async_copy(vals_v, pen_ref.at[flat_v.at[0]], sem).wait()


@functools.cache
def _make_sc_fixup():
    mesh = plsc.VectorSubcoreMesh(core_axis_name="c", subcore_axis_name="s")
    return pl.kernel(
        _sc_fixup_body,
        out_type=(),
        mesh=mesh,
        scratch_types=[
            pltpu.VMEM((B,), jnp.int32),
            pltpu.VMEM((1, B), jnp.int32),
            pltpu.VMEM((B,), jnp.float32),
            pltpu.VMEM((L,), jnp.float32),
            pltpu.SemaphoreType.DMA,
        ],
    )


def kernel(logits, repeat_penality, penality_value):
    log_t = logits.T
    pen_t = repeat_penality.T
    idx8, out_t = _stream_pass(log_t, pen_t)
    idx = idx8[0]
    pen_flat_ref = jax.new_ref(out_t.reshape(V * B))
    pv16 = jnp.full((L,), penality_value, dtype=jnp.float32)
    _make_sc_fixup()(pen_flat_ref, idx, pv16)
    return idx.reshape(B, 1), pen_flat_ref[...].reshape(V, B).T
